# baseline (speedup 1.0000x reference)
import functools

import jax
import jax.numpy as jnp
from jax.experimental import pallas as pl
from jax.experimental.pallas import tpu as pltpu

N = 8192
IN = 128
H1 = 64
H2 = 64
C = 16

BLK = 256


def _coatt_kernel(pack_ref, sadj_ref, fadj_ref, g1b_ref, g2b_ref,
                  vw_ref, vb_ref, dw_ref, db_ref, rb_ref,
                  out_ref, p1_ref, p2_ref, c_ref):
    i = pl.program_id(0)

    @pl.when(i == 0)
    def _prologue():
        f32 = jnp.float32
        g1w = pack_ref[N:N + IN, 0:H1]
        g2w = pack_ref[N + IN:N + 2 * IN, 0:H1]
        rw = pack_ref[N + 2 * IN:N + 2 * IN + H2, 0:C]
        vd = jnp.dot(vw_ref[...], dw_ref[...], preferred_element_type=f32)
        m = jnp.dot(vd, rw, preferred_element_type=f32)
        w1 = jnp.dot(g1w, m, preferred_element_type=f32)
        w2 = jnp.dot(g2w, m, preferred_element_type=f32)
        p1_ref[...] = jnp.dot(pack_ref[0:N, :], w1, preferred_element_type=f32)
        p2_ref[...] = jnp.dot(pack_ref[0:N, :], w2, preferred_element_type=f32)
        gb = 0.5 * (g1b_ref[...] + g2b_ref[...])[None, :]
        vb_d = jnp.dot(vb_ref[...][None, :], dw_ref[...],
                       preferred_element_type=f32)
        c_ref[...] = (jnp.dot(gb, m, preferred_element_type=f32)
                      + jnp.dot(vb_d + db_ref[...][None, :], rw,
                                preferred_element_type=f32)
                      + rb_ref[...][None, :])

    acc = jnp.dot(sadj_ref[...], p1_ref[...],
                  preferred_element_type=jnp.float32,
                  precision=jax.lax.Precision.DEFAULT)
    acc = acc + jnp.dot(fadj_ref[...], p2_ref[...],
                        preferred_element_type=jnp.float32,
                        precision=jax.lax.Precision.DEFAULT)
    res = 0.5 * acc + c_ref[...]
    mx = jnp.max(res, axis=1, keepdims=True)
    lse = jnp.log(jnp.sum(jnp.exp(res - mx), axis=1, keepdims=True)) + mx
    out_ref[...] = res - lse


@functools.partial(jax.jit, static_argnames=())
def _run(input_feature, sadj, fadj, gcn1_w, gcn1_b, gcn2_w, gcn2_b,
         v_w, v_b, dense_w, dense_b, res_w, res_b):
    nblk = pl.cdiv(N, BLK)
    npack = N + 2 * IN + H2
    pack = jnp.concatenate(
        [input_feature,
         jnp.pad(gcn1_w, ((0, 0), (0, IN - H1))),
         jnp.pad(gcn2_w, ((0, 0), (0, IN - H1))),
         jnp.pad(res_w, ((0, 0), (0, IN - C)))], axis=0)
    full = lambda shape: pl.BlockSpec(shape, lambda i: (0,) * len(shape))
    return pl.pallas_call(
        _coatt_kernel,
        grid=(nblk,),
        in_specs=[
            full((npack, IN)),
            pl.BlockSpec((BLK, N), lambda i: (i, 0)),
            pl.BlockSpec((BLK, N), lambda i: (i, 0)),
            full((H1,)),
            full((H1,)),
            full((H1, H2)),
            full((H2,)),
            full((H2, H2)),
            full((H2,)),
            full((C,)),
        ],
        out_specs=pl.BlockSpec((BLK, C), lambda i: (i, 0)),
        scratch_shapes=[
            pltpu.VMEM((N, C), jnp.float32),
            pltpu.VMEM((N, C), jnp.float32),
            pltpu.VMEM((1, C), jnp.float32),
        ],
        out_shape=jax.ShapeDtypeStruct((N, C), jnp.float32),
        compiler_params=pltpu.CompilerParams(
            dimension_semantics=("arbitrary",),
            vmem_limit_bytes=63 * 1024 * 1024,
        ),
    )(pack, sadj, fadj, gcn1_b, gcn2_b, v_w, v_b, dense_w, dense_b, res_b)


def kernel(input_feature, sadj, fadj, gcn1_w, gcn1_b, gcn2_w, gcn2_b, q_w,
           q_b, k_w, k_b, v_w, v_b, dense_w, dense_b, res_w, res_b):
    return _run(input_feature, sadj, fadj, gcn1_w, gcn1_b, gcn2_w, gcn2_b,
                v_w, v_b, dense_w, dense_b, res_w, res_b)


# speedup vs baseline: 1.0202x; 1.0202x over previous
import functools

import jax
import jax.numpy as jnp
from jax.experimental import pallas as pl
from jax.experimental.pallas import tpu as pltpu

N = 8192
IN = 128
H1 = 64
H2 = 64
C = 16

BLK = 256


def _coatt_kernel(x_ref, sadj_ref, fadj_ref, g1w_ref, g1b_ref, g2w_ref,
                  g2b_ref, vw_ref, vb_ref, dw_ref, db_ref, rw_ref, rb_ref,
                  out_ref, p1_ref, p2_ref, c_ref):
    i = pl.program_id(0)

    @pl.when(i == 0)
    def _prologue():
        f32 = jnp.float32
        rw = rw_ref[...]
        vd = jnp.dot(vw_ref[...], dw_ref[...], preferred_element_type=f32)
        m = jnp.dot(vd, rw, preferred_element_type=f32)
        w1 = jnp.dot(g1w_ref[...], m, preferred_element_type=f32)
        w2 = jnp.dot(g2w_ref[...], m, preferred_element_type=f32)
        p1_ref[...] = jnp.dot(x_ref[...], w1, preferred_element_type=f32)
        p2_ref[...] = jnp.dot(x_ref[...], w2, preferred_element_type=f32)
        gb = 0.5 * (g1b_ref[...] + g2b_ref[...])[None, :]
        vb_d = jnp.dot(vb_ref[...][None, :], dw_ref[...],
                       preferred_element_type=f32)
        c_ref[...] = (jnp.dot(gb, m, preferred_element_type=f32)
                      + jnp.dot(vb_d + db_ref[...][None, :], rw,
                                preferred_element_type=f32)
                      + rb_ref[...][None, :])

    acc = jnp.dot(sadj_ref[...], p1_ref[...],
                  preferred_element_type=jnp.float32,
                  precision=jax.lax.Precision.DEFAULT)
    acc = acc + jnp.dot(fadj_ref[...], p2_ref[...],
                        preferred_element_type=jnp.float32,
                        precision=jax.lax.Precision.DEFAULT)
    res = 0.5 * acc + c_ref[...]
    mx = jnp.max(res, axis=1, keepdims=True)
    lse = jnp.log(jnp.sum(jnp.exp(res - mx), axis=1, keepdims=True)) + mx
    out_ref[...] = res - lse


@functools.partial(jax.jit, static_argnames=())
def _run(input_feature, sadj, fadj, gcn1_w, gcn1_b, gcn2_w, gcn2_b,
         v_w, v_b, dense_w, dense_b, res_w, res_b):
    nblk = pl.cdiv(N, BLK)
    full = lambda shape: pl.BlockSpec(shape, lambda i: (0,) * len(shape))
    return pl.pallas_call(
        _coatt_kernel,
        grid=(nblk,),
        in_specs=[
            full((N, IN)),
            pl.BlockSpec((BLK, N), lambda i: (i, 0)),
            pl.BlockSpec((BLK, N), lambda i: (i, 0)),
            full((IN, H1)),
            full((H1,)),
            full((IN, H1)),
            full((H1,)),
            full((H1, H2)),
            full((H2,)),
            full((H2, H2)),
            full((H2,)),
            full((H2, C)),
            full((C,)),
        ],
        out_specs=pl.BlockSpec((BLK, C), lambda i: (i, 0)),
        scratch_shapes=[
            pltpu.VMEM((N, C), jnp.float32),
            pltpu.VMEM((N, C), jnp.float32),
            pltpu.VMEM((1, C), jnp.float32),
        ],
        out_shape=jax.ShapeDtypeStruct((N, C), jnp.float32),
        compiler_params=pltpu.CompilerParams(
            dimension_semantics=("arbitrary",),
            vmem_limit_bytes=63 * 1024 * 1024,
        ),
    )(input_feature, sadj, fadj, gcn1_w, gcn1_b, gcn2_w, gcn2_b,
      v_w, v_b, dense_w, dense_b, res_w, res_b)


def kernel(input_feature, sadj, fadj, gcn1_w, gcn1_b, gcn2_w, gcn2_b, q_w,
           q_b, k_w, k_b, v_w, v_b, dense_w, dense_b, res_w, res_b):
    return _run(input_feature, sadj, fadj, gcn1_w, gcn1_b, gcn2_w, gcn2_b,
                v_w, v_b, dense_w, dense_b, res_w, res_b)


# speedup vs baseline: 1.0224x; 1.0021x over previous
import functools

import jax
import jax.numpy as jnp
from jax.experimental import pallas as pl
from jax.experimental.pallas import tpu as pltpu

N = 8192
IN = 128
H1 = 64
H2 = 64
C = 16

BLK = 256


def _coatt_kernel(x_ref, sadj_ref, fadj_ref, g1w_ref, g1b_ref, g2w_ref,
                  g2b_ref, vw_ref, vb_ref, dw_ref, db_ref, rw_ref, rb_ref,
                  out_ref, p1_ref, p2_ref, c_ref):
    i = pl.program_id(0)

    @pl.when(i == 0)
    def _prologue():
        f32 = jnp.float32
        vd = jnp.dot(vw_ref[...], dw_ref[...], preferred_element_type=f32)
        m = jnp.dot(vd, rw_ref[...], preferred_element_type=f32)
        w1 = jnp.dot(g1w_ref[...], m, preferred_element_type=f32)
        w2 = jnp.dot(g2w_ref[...], m, preferred_element_type=f32)
        p1_ref[...] = jnp.dot(x_ref[...], w1, preferred_element_type=f32)
        p2_ref[...] = jnp.dot(x_ref[...], w2, preferred_element_type=f32)
        gb = 0.5 * (g1b_ref[...] + g2b_ref[...])
        vb_d = jnp.dot(vb_ref[...], dw_ref[...], preferred_element_type=f32)
        c_ref[...] = (jnp.dot(gb, m, preferred_element_type=f32)
                      + jnp.dot(vb_d + db_ref[...], rw_ref[...],
                                preferred_element_type=f32)
                      + rb_ref[...])

    acc = jnp.dot(sadj_ref[...], p1_ref[...],
                  preferred_element_type=jnp.float32,
                  precision=jax.lax.Precision.DEFAULT)
    acc = acc + jnp.dot(fadj_ref[...], p2_ref[...],
                        preferred_element_type=jnp.float32,
                        precision=jax.lax.Precision.DEFAULT)
    res = 0.5 * acc + c_ref[...]
    mx = jnp.max(res, axis=1, keepdims=True)
    lse = jnp.log(jnp.sum(jnp.exp(res - mx), axis=1, keepdims=True)) + mx
    out_ref[...] = res - lse


@functools.partial(jax.jit, static_argnames=())
def _run(input_feature, sadj, fadj, gcn1_w, gcn1_b, gcn2_w, gcn2_b,
         v_w, v_b, dense_w, dense_b, res_w, res_b):
    nblk = pl.cdiv(N, BLK)
    full = lambda shape: pl.BlockSpec(shape, lambda i: (0,) * len(shape))
    return pl.pallas_call(
        _coatt_kernel,
        grid=(nblk,),
        in_specs=[
            full((N, IN)),
            pl.BlockSpec((BLK, N), lambda i: (i, 0)),
            pl.BlockSpec((BLK, N), lambda i: (i, 0)),
            full((IN, H1)),
            full((1, H1)),
            full((IN, H1)),
            full((1, H1)),
            full((H1, H2)),
            full((1, H2)),
            full((H2, H2)),
            full((1, H2)),
            full((H2, C)),
            full((1, C)),
        ],
        out_specs=pl.BlockSpec((BLK, C), lambda i: (i, 0)),
        scratch_shapes=[
            pltpu.VMEM((N, C), jnp.float32),
            pltpu.VMEM((N, C), jnp.float32),
            pltpu.VMEM((1, C), jnp.float32),
        ],
        out_shape=jax.ShapeDtypeStruct((N, C), jnp.float32),
        compiler_params=pltpu.CompilerParams(
            dimension_semantics=("arbitrary",),
            vmem_limit_bytes=63 * 1024 * 1024,
        ),
    )(input_feature, sadj, fadj, gcn1_w, gcn1_b.reshape(1, H1), gcn2_w,
      gcn2_b.reshape(1, H1), v_w, v_b.reshape(1, H2), dense_w,
      dense_b.reshape(1, H2), res_w, res_b.reshape(1, C))


def kernel(input_feature, sadj, fadj, gcn1_w, gcn1_b, gcn2_w, gcn2_b, q_w,
           q_b, k_w, k_b, v_w, v_b, dense_w, dense_b, res_w, res_b):
    return _run(input_feature, sadj, fadj, gcn1_w, gcn1_b, gcn2_w, gcn2_b,
                v_w, v_b, dense_w, dense_b, res_w, res_b)


# speedup vs baseline: 1.0351x; 1.0124x over previous
import functools

import jax
import jax.numpy as jnp
from jax.experimental import pallas as pl
from jax.experimental.pallas import tpu as pltpu

N = 8192
IN = 128
H1 = 64
H2 = 64
C = 16

BLK = 256


def _coatt_kernel(x_ref, sadj_ref, fadj_ref, g1w_ref, g1b_ref, g2w_ref,
                  g2b_ref, vw_ref, vb_ref, dw_ref, db_ref, rw_ref, rb_ref,
                  out_ref, p1_ref, p2_ref, c_ref):
    i = pl.program_id(0)

    @pl.when(i == 0)
    def _prologue():
        f32 = jnp.float32
        vd = jnp.dot(vw_ref[...], dw_ref[...], preferred_element_type=f32)
        m = jnp.dot(vd, rw_ref[...], preferred_element_type=f32)
        w1 = jnp.dot(g1w_ref[...], m, preferred_element_type=f32)
        w2 = jnp.dot(g2w_ref[...], m, preferred_element_type=f32)
        dn_t = (((0,), (1,)), ((), ()))
        p1_ref[...] = jax.lax.dot_general(w1, x_ref[...], dn_t,
                                          preferred_element_type=f32)
        p2_ref[...] = jax.lax.dot_general(w2, x_ref[...], dn_t,
                                          preferred_element_type=f32)
        gb = 0.5 * (g1b_ref[...] + g2b_ref[...])
        vb_d = jnp.dot(vb_ref[...], dw_ref[...], preferred_element_type=f32)
        c_ref[...] = (jnp.dot(gb, m, preferred_element_type=f32)
                      + jnp.dot(vb_d + db_ref[...], rw_ref[...],
                                preferred_element_type=f32)
                      + rb_ref[...])

    dn = (((1,), (1,)), ((), ()))
    acc = jax.lax.dot_general(sadj_ref[...], p1_ref[...], dn,
                              preferred_element_type=jnp.float32)
    acc = acc + jax.lax.dot_general(fadj_ref[...], p2_ref[...], dn,
                                    preferred_element_type=jnp.float32)
    res = 0.5 * acc + c_ref[...]
    mx = jnp.max(res, axis=1, keepdims=True)
    lse = jnp.log(jnp.sum(jnp.exp(res - mx), axis=1, keepdims=True)) + mx
    out_ref[...] = res - lse


@functools.partial(jax.jit, static_argnames=())
def _run(input_feature, sadj, fadj, gcn1_w, gcn1_b, gcn2_w, gcn2_b,
         v_w, v_b, dense_w, dense_b, res_w, res_b):
    nblk = pl.cdiv(N, BLK)
    full = lambda shape: pl.BlockSpec(shape, lambda i: (0,) * len(shape))
    return pl.pallas_call(
        _coatt_kernel,
        grid=(nblk,),
        in_specs=[
            full((N, IN)),
            pl.BlockSpec((BLK, N), lambda i: (i, 0)),
            pl.BlockSpec((BLK, N), lambda i: (i, 0)),
            full((IN, H1)),
            full((1, H1)),
            full((IN, H1)),
            full((1, H1)),
            full((H1, H2)),
            full((1, H2)),
            full((H2, H2)),
            full((1, H2)),
            full((H2, C)),
            full((1, C)),
        ],
        out_specs=pl.BlockSpec((BLK, C), lambda i: (i, 0)),
        scratch_shapes=[
            pltpu.VMEM((C, N), jnp.float32),
            pltpu.VMEM((C, N), jnp.float32),
            pltpu.VMEM((1, C), jnp.float32),
        ],
        out_shape=jax.ShapeDtypeStruct((N, C), jnp.float32),
        compiler_params=pltpu.CompilerParams(
            dimension_semantics=("arbitrary",),
            vmem_limit_bytes=63 * 1024 * 1024,
        ),
    )(input_feature, sadj, fadj, gcn1_w, gcn1_b.reshape(1, H1), gcn2_w,
      gcn2_b.reshape(1, H1), v_w, v_b.reshape(1, H2), dense_w,
      dense_b.reshape(1, H2), res_w, res_b.reshape(1, C))


def kernel(input_feature, sadj, fadj, gcn1_w, gcn1_b, gcn2_w, gcn2_b, q_w,
           q_b, k_w, k_b, v_w, v_b, dense_w, dense_b, res_w, res_b):
    return _run(input_feature, sadj, fadj, gcn1_w, gcn1_b, gcn2_w, gcn2_b,
                v_w, v_b, dense_w, dense_b, res_w, res_b)
